# baseline (speedup 1.0000x reference)
import functools

import jax
import jax.numpy as jnp
from jax import lax
from jax.experimental import pallas as pl
from jax.experimental.pallas import tpu as pltpu
from jax.experimental.pallas import tpu_sc as plsc

N = 10000
E = 320000
D = 128
ED = 16
B = 64
EMB = 16

NC = 2
NS = 16
NW = NC * NS
EW = E // NW
C = 80
NCHUNK = EW // C
ROWS_PER_TILE = 632


DEPTH = 2


NPHASE = 5
CPP = NCHUNK // NPHASE
EPP = EW // NPHASE


def _sc_mp_body(h_hbm, ea_hbm, src_hbm, dst_hbm, zero_hbm, out_hbm,
                agg_sh, src_v, dst_v, rows, eab,
                gsems, easems, ssems):
    c = lax.axis_index("c")
    s_id = lax.axis_index("s")
    w = s_id * NC + c

    row0 = pl.multiple_of(
        jnp.minimum(s_id * ROWS_PER_TILE, N - ROWS_PER_TILE), 8)
    pltpu.sync_copy(zero_hbm.at[pl.ds(row0, ROWS_PER_TILE)],
                    agg_sh.at[pl.ds(row0, ROWS_PER_TILE)])
    plsc.subcore_barrier()

    for half in range(NPHASE):
        ebase = w * EW + half * EPP
        pltpu.sync_copy(src_hbm.at[w, half, 0], src_v)
        pltpu.sync_copy(dst_hbm.at[w, half], dst_v)

        def issue_loads(k, slot, _src=src_v, _eb=ebase):
            off = pl.multiple_of(k * C, 8)
            pltpu.async_copy(h_hbm.at[_src.at[pl.ds(off, C)]], rows[slot],
                             gsems[slot])
            eoff = pl.multiple_of(_eb + k * C, 8)
            pltpu.async_copy(ea_hbm.at[pl.ds(eoff, C)], eab[slot],
                             easems[slot])

        def wait_loads(slot):
            pltpu.make_async_copy(h_hbm.at[pl.ds(0, C)], rows[slot],
                                  gsems[slot]).wait()
            pltpu.make_async_copy(ea_hbm.at[pl.ds(0, C)], eab[slot],
                                  easems[slot]).wait()

        def compute(slot):
            def rowfn(r, rc):
                for j in range(D // 16):
                    sl = pl.ds(j * 16, 16)
                    eab[slot][r, sl] = jnp.maximum(
                        rows[slot][r, sl] + eab[slot][r, sl], 0.0)
                return rc

            lax.fori_loop(0, C, rowfn, 0)

        def scatter(k, slot):
            pltpu.async_copy(eab[slot], agg_sh.at[dst_v.at[k]], ssems[slot],
                             add=True)
            pltpu.make_async_copy(h_hbm.at[pl.ds(0, C)], eab[slot],
                                  ssems[slot]).wait()

        def process(k, slot, prefetch):
            wait_loads(slot)
            compute(slot)
            if prefetch:
                pltpu.async_copy(
                    h_hbm.at[src_v.at[pl.ds(pl.multiple_of((k + DEPTH) * C, 8),
                                            C)]],
                    rows[slot], gsems[slot])
            scatter(k, slot)
            if prefetch:
                eoff = pl.multiple_of(ebase + (k + DEPTH) * C, 8)
                pltpu.async_copy(ea_hbm.at[pl.ds(eoff, C)], eab[slot],
                                 easems[slot])

        for s in range(DEPTH):
            issue_loads(s, s)

        nu = CPP - DEPTH
        nb = nu // DEPTH

        def outer(ko, carry):
            for s in range(DEPTH):
                process(ko * DEPTH + s, s, True)
            return carry

        lax.fori_loop(0, nb, outer, 0)
        for k in range(nb * DEPTH, CPP):
            process(k, k % DEPTH, k < nu)

    plsc.subcore_barrier()

    pltpu.sync_copy(agg_sh.at[pl.ds(row0, ROWS_PER_TILE)],
                    out_hbm.at[c, pl.ds(row0, ROWS_PER_TILE)])


@functools.cache
def _make_sc_message_pass():
    sem = pltpu.SemaphoreType.DMA
    return pl.kernel(
        _sc_mp_body,
        out_type=jax.ShapeDtypeStruct((NC, N, D), jnp.float32),
        mesh=plsc.VectorSubcoreMesh(core_axis_name="c", subcore_axis_name="s"),
        scratch_types=[
            pltpu.VMEM_SHARED((N, D), jnp.float32),
            pltpu.VMEM((EPP,), jnp.int32),
            pltpu.VMEM((CPP, C), jnp.int32),
            tuple(pltpu.VMEM((C, D), jnp.float32) for _ in range(DEPTH)),
            tuple(pltpu.VMEM((C, D), jnp.float32) for _ in range(DEPTH)),
            tuple(sem for _ in range(DEPTH)),
            tuple(sem for _ in range(DEPTH)),
            tuple(sem for _ in range(DEPTH)),
        ],
    )


BE = 5000
BN = 2000


def _ea_one_body(eat_ref, we, be_, o):
    a = eat_ref[...]
    o[...] = jnp.dot(a, we[...], preferred_element_type=jnp.float32, precision=lax.Precision.HIGHEST) + be_[...]


def _ea_one(edge_attr, Wei, bei):
    full = lambda shape: pl.BlockSpec(shape, lambda i: (0,) * len(shape))
    return pl.pallas_call(
        _ea_one_body,
        grid=(E // BE,),
        in_specs=[
            pl.BlockSpec((BE, ED), lambda i: (i, 0)),
            full((ED, D)), full((1, D)),
        ],
        out_specs=pl.BlockSpec((BE, D), lambda i: (i, 0)),
        out_shape=jax.ShapeDtypeStruct((E, D), jnp.float32),
    )(edge_attr, Wei, bei.reshape(1, D))


def _mlp_body(h_ref, a0_ref, a1_ref, w1, b1, w2, b2, sc_ref, wg1, bg1, wg2,
              bg2, out_ref, gate_ref, *, relu_last, with_gate):
    hin = h_ref[...] * sc_ref[0, 0] + a0_ref[0] + a1_ref[0]
    t = jnp.maximum(jnp.dot(hin, w1[...], preferred_element_type=jnp.float32, precision=lax.Precision.HIGHEST)
                    + b1[...], 0.0)
    o = jnp.dot(t, w2[...], preferred_element_type=jnp.float32, precision=lax.Precision.HIGHEST) + b2[...]
    if relu_last:
        o = jnp.maximum(o, 0.0)
    out_ref[...] = o
    if with_gate:
        g = jnp.maximum(jnp.dot(o, wg1[...], preferred_element_type=jnp.float32, precision=lax.Precision.HIGHEST)
                        + bg1[...], 0.0)
        gate_ref[...] = jnp.dot(g, wg2[...], preferred_element_type=jnp.float32, precision=lax.Precision.HIGHEST) + bg2[...]
    else:
        gate_ref[...] = jnp.zeros_like(gate_ref)


def _mlp(h, parts, W1i, b1i, W2i, b2i, scale, Wg1, bg1, Wg2, bg2, relu_last,
         with_gate):
    full = lambda shape: pl.BlockSpec(shape, lambda i: (0,) * len(shape))
    out, gate = pl.pallas_call(
        functools.partial(_mlp_body, relu_last=relu_last, with_gate=with_gate),
        grid=(N // BN,),
        in_specs=[
            pl.BlockSpec((BN, D), lambda i: (i, 0)),
            pl.BlockSpec((1, BN, D), lambda i: (0, i, 0)),
            pl.BlockSpec((1, BN, D), lambda i: (1, i, 0)),
            full((D, D)), full((1, D)), full((D, D)), full((1, D)),
            full((1, 1)),
            full((D, D)), full((1, D)), full((D, 1)), full((1, 1)),
        ],
        out_specs=[pl.BlockSpec((BN, D), lambda i: (i, 0)),
                   pl.BlockSpec((BN, 1), lambda i: (i, 0))],
        out_shape=[jax.ShapeDtypeStruct((N, D), jnp.float32),
                   jax.ShapeDtypeStruct((N, 1), jnp.float32)],
    )(h, parts, parts, W1i, b1i.reshape(1, D), W2i, b2i.reshape(1, D),
      scale, Wg1, bg1.reshape(1, D), Wg2, bg2.reshape(1, 1))
    return out, gate


def _seg_max_body(gate_ref, batch_ref, out_ref):
    i = pl.program_id(0)
    mask = batch_ref[...] == lax.broadcasted_iota(jnp.int32, (BN, B), 1)
    vals = jnp.where(mask, gate_ref[...], -jnp.inf)
    cur = jnp.max(vals, axis=0, keepdims=True)

    @pl.when(i == 0)
    def _():
        out_ref[...] = cur

    @pl.when(i > 0)
    def _():
        out_ref[...] = jnp.maximum(out_ref[...], cur)


def _seg_max(gate, batch2d):
    return pl.pallas_call(
        _seg_max_body,
        grid=(N // BN,),
        in_specs=[pl.BlockSpec((BN, 1), lambda i: (i, 0)),
                  pl.BlockSpec((BN, 1), lambda i: (i, 0))],
        out_specs=pl.BlockSpec((1, B), lambda i: (0, 0)),
        out_shape=jax.ShapeDtypeStruct((1, B), jnp.float32),
    )(gate, batch2d)


def _seg_accum_body(gate_ref, batch_ref, gmax_ref, h_ref, num_ref, gsum_ref):
    i = pl.program_id(0)
    gm = gmax_ref[...]
    gm = jnp.where(jnp.isfinite(gm), gm, 0.0)
    mask = batch_ref[...] == lax.broadcasted_iota(jnp.int32, (BN, B), 1)
    gm_node = jnp.sum(jnp.where(mask, gm, 0.0), axis=1, keepdims=True)
    ge = jnp.exp(gate_ref[...] - gm_node)
    wmask = mask.astype(jnp.float32) * ge
    ones = jnp.ones((BN, 1), jnp.float32)
    gsum_cur = lax.dot_general(wmask, ones, (((0,), (0,)), ((), ())),
                               preferred_element_type=jnp.float32,
                               precision=lax.Precision.HIGHEST)
    num_cur = lax.dot_general(wmask, h_ref[...], (((0,), (0,)), ((), ())),
                              preferred_element_type=jnp.float32,
                              precision=lax.Precision.HIGHEST)

    @pl.when(i == 0)
    def _():
        num_ref[...] = num_cur
        gsum_ref[...] = gsum_cur

    @pl.when(i > 0)
    def _():
        num_ref[...] = num_ref[...] + num_cur
        gsum_ref[...] = gsum_ref[...] + gsum_cur


def _seg_accum(gate, batch2d, gmax, h):
    return pl.pallas_call(
        _seg_accum_body,
        grid=(N // BN,),
        in_specs=[pl.BlockSpec((BN, 1), lambda i: (i, 0)),
                  pl.BlockSpec((BN, 1), lambda i: (i, 0)),
                  pl.BlockSpec((1, B), lambda i: (0, 0)),
                  pl.BlockSpec((BN, D), lambda i: (i, 0))],
        out_specs=[pl.BlockSpec((B, D), lambda i: (0, 0)),
                   pl.BlockSpec((B, 1), lambda i: (0, 0))],
        out_shape=[jax.ShapeDtypeStruct((B, D), jnp.float32),
                   jax.ShapeDtypeStruct((B, 1), jnp.float32)],
    )(gate, batch2d, gmax, h)


def _head_body(num_ref, gsum_ref, il, ia, ib, iar, el, ea_, eb, ear,
               wp, wl, wa, wb, war, bl1_ref, wm, bm_ref, out_ref):
    pooled = num_ref[...] / (gsum_ref[...] + 1e-16)

    def onehot(idx_ref, k):
        return (idx_ref[...] == lax.broadcasted_iota(jnp.int32, (B, k), 1)
                ).astype(jnp.float32)

    dot = lambda a_, b_: jnp.dot(a_, b_, preferred_element_type=jnp.float32,
                                 precision=lax.Precision.HIGHEST)
    z = dot(pooled, wp[...])
    z = z + dot(dot(onehot(il, 16), el[...]), wl[...])
    z = z + dot(dot(onehot(ia, 24), ea_[...]), wa[...])
    z = z + dot(dot(onehot(ib, 4), eb[...]), wb[...])
    z = z + dot(dot(onehot(iar, 16), ear[...]), war[...])
    z = jnp.maximum(z + bl1_ref[...], 0.0)
    out_ref[...] = dot(z, wm[...]) + bm_ref[...]


def _head(num, gsum, ligand_idx, additive_idx, base_idx, aryl_idx,
          emb_ligand, emb_additive, emb_base, emb_aryl, Wl1, bl1, Wm, bm):
    args = (num, gsum, ligand_idx.reshape(B, 1), additive_idx.reshape(B, 1),
            base_idx.reshape(B, 1), aryl_idx.reshape(B, 1),
            emb_ligand, emb_additive, emb_base, emb_aryl,
            Wl1[:D], Wl1[D:D + EMB], Wl1[D + EMB:D + 2 * EMB],
            Wl1[D + 2 * EMB:D + 3 * EMB], Wl1[D + 3 * EMB:],
            bl1.reshape(1, D), Wm, bm.reshape(1, 1))
    return pl.pallas_call(
        _head_body,
        out_shape=jax.ShapeDtypeStruct((B, 1), jnp.float32),
    )(*args)


def kernel(x, edge_index, batch, ligand_idx, additive_idx, base_idx, aryl_idx,
           edge_attr, W1, b1, W2, b2, We, be, eps, Wg1, bg1, Wg2, bg2,
           emb_ligand, emb_additive, emb_base, emb_aryl, Wl1, bl1, Wm, bm):
    src = edge_index[0].reshape(NW, NPHASE, 1, EPP)
    dst = edge_index[1].reshape(NW, NPHASE, CPP, C)
    zeros = jnp.zeros((N, D), jnp.float32)
    batch2d = batch.reshape(N, 1)
    scales = (1.0 + eps).reshape(3, 1, 1)

    sc_mp = _make_sc_message_pass()

    h = x
    gate = None
    for i in range(3):
        ea_i = _ea_one(edge_attr, We[i], be[i])
        parts = sc_mp(h, ea_i, src, dst, zeros)
        h, gate = _mlp(h, parts, W1[i], b1[i], W2[i], b2[i], scales[i],
                       Wg1, bg1, Wg2, bg2, relu_last=(i < 2),
                       with_gate=(i == 2))

    gmax = _seg_max(gate, batch2d)
    num, gsum = _seg_accum(gate, batch2d, gmax, h)
    return _head(num, gsum, ligand_idx, additive_idx, base_idx, aryl_idx,
                 emb_ligand, emb_additive, emb_base, emb_aryl, Wl1, bl1,
                 Wm, bm)


# speedup vs baseline: 1.2128x; 1.2128x over previous
import functools

import jax
import jax.numpy as jnp
from jax import lax
from jax.experimental import pallas as pl
from jax.experimental.pallas import tpu as pltpu
from jax.experimental.pallas import tpu_sc as plsc

N = 10000
E = 320000
D = 128
ED = 16
B = 64
EMB = 16

NC = 2
NS = 16
NW = NC * NS
EW = E // NW
C = 80
NCHUNK = EW // C
ROWS_PER_TILE = 632


DEPTH = 2


NPHASE = 5
CPP = NCHUNK // NPHASE
EPP = EW // NPHASE


def _sc_mp_body(h_hbm, ea_hbm, src_hbm, dst_hbm, zero_hbm, out_hbm,
                agg_sh, src_v, dst_v, rows, eab,
                gsems, easems, ssems):
    c = lax.axis_index("c")
    s_id = lax.axis_index("s")
    w = s_id * NC + c

    row0 = pl.multiple_of(
        jnp.minimum(s_id * ROWS_PER_TILE, N - ROWS_PER_TILE), 8)
    pltpu.sync_copy(zero_hbm.at[pl.ds(row0, ROWS_PER_TILE)],
                    agg_sh.at[pl.ds(row0, ROWS_PER_TILE)])
    plsc.subcore_barrier()

    for half in range(NPHASE):
        ebase = w * EW + half * EPP
        pltpu.sync_copy(src_hbm.at[w, half, 0], src_v)
        pltpu.sync_copy(dst_hbm.at[w, half], dst_v)

        def issue_loads(k, slot, _src=src_v, _eb=ebase):
            off = pl.multiple_of(k * C, 8)
            pltpu.async_copy(h_hbm.at[_src.at[pl.ds(off, C)]], rows[slot],
                             gsems[slot])
            eoff = pl.multiple_of(_eb + k * C, 8)
            pltpu.async_copy(ea_hbm.at[pl.ds(eoff, C)], eab[slot],
                             easems[slot])

        def wait_loads(slot):
            pltpu.make_async_copy(h_hbm.at[pl.ds(0, C)], rows[slot],
                                  gsems[slot]).wait()
            pltpu.make_async_copy(ea_hbm.at[pl.ds(0, C)], eab[slot],
                                  easems[slot]).wait()

        def compute(slot):
            def rowfn(r, rc):
                for j in range(D // 16):
                    sl = pl.ds(j * 16, 16)
                    eab[slot][r, sl] = jnp.maximum(
                        rows[slot][r, sl] + eab[slot][r, sl], 0.0)
                return rc

            lax.fori_loop(0, C, rowfn, 0)

        def scatter(k, slot):
            pltpu.async_copy(eab[slot], agg_sh.at[dst_v.at[k]], ssems[slot],
                             add=True)
            pltpu.make_async_copy(h_hbm.at[pl.ds(0, C)], eab[slot],
                                  ssems[slot]).wait()

        def process(k, slot, prefetch):
            wait_loads(slot)
            compute(slot)
            if prefetch:
                pltpu.async_copy(
                    h_hbm.at[src_v.at[pl.ds(pl.multiple_of((k + DEPTH) * C, 8),
                                            C)]],
                    rows[slot], gsems[slot])
            scatter(k, slot)
            if prefetch:
                eoff = pl.multiple_of(ebase + (k + DEPTH) * C, 8)
                pltpu.async_copy(ea_hbm.at[pl.ds(eoff, C)], eab[slot],
                                 easems[slot])

        for s in range(DEPTH):
            issue_loads(s, s)

        nu = CPP - DEPTH
        nb = nu // DEPTH

        def outer(ko, carry):
            for s in range(DEPTH):
                process(ko * DEPTH + s, s, True)
            return carry

        lax.fori_loop(0, nb, outer, 0)
        for k in range(nb * DEPTH, CPP):
            process(k, k % DEPTH, k < nu)

    plsc.subcore_barrier()

    pltpu.sync_copy(agg_sh.at[pl.ds(row0, ROWS_PER_TILE)],
                    out_hbm.at[c, pl.ds(row0, ROWS_PER_TILE)])


@functools.cache
def _make_sc_message_pass():
    sem = pltpu.SemaphoreType.DMA
    return pl.kernel(
        _sc_mp_body,
        out_type=jax.ShapeDtypeStruct((NC, N, D), jnp.float32),
        mesh=plsc.VectorSubcoreMesh(core_axis_name="c", subcore_axis_name="s"),
        scratch_types=[
            pltpu.VMEM_SHARED((N, D), jnp.float32),
            pltpu.VMEM((EPP,), jnp.int32),
            pltpu.VMEM((CPP, C), jnp.int32),
            tuple(pltpu.VMEM((C, D), jnp.float32) for _ in range(DEPTH)),
            tuple(pltpu.VMEM((C, D), jnp.float32) for _ in range(DEPTH)),
            tuple(sem for _ in range(DEPTH)),
            tuple(sem for _ in range(DEPTH)),
            tuple(sem for _ in range(DEPTH)),
        ],
    )


BE = 5000
BN = 2000


def _ea_one_body(eat_ref, we, be_, o):
    a = eat_ref[...]
    o[...] = jnp.dot(a, we[...], preferred_element_type=jnp.float32) + be_[...]


def _ea_one(edge_attr, Wei, bei):
    full = lambda shape: pl.BlockSpec(shape, lambda i: (0,) * len(shape))
    return pl.pallas_call(
        _ea_one_body,
        grid=(E // BE,),
        in_specs=[
            pl.BlockSpec((BE, ED), lambda i: (i, 0)),
            full((ED, D)), full((1, D)),
        ],
        out_specs=pl.BlockSpec((BE, D), lambda i: (i, 0)),
        out_shape=jax.ShapeDtypeStruct((E, D), jnp.float32),
    )(edge_attr, Wei, bei.reshape(1, D))


def _mlp_body(h_ref, a0_ref, a1_ref, w1, b1, w2, b2, sc_ref, wg1, bg1, wg2,
              bg2, out_ref, gate_ref, *, relu_last, with_gate):
    hin = h_ref[...] * sc_ref[0, 0] + a0_ref[0] + a1_ref[0]
    t = jnp.maximum(jnp.dot(hin, w1[...], preferred_element_type=jnp.float32)
                    + b1[...], 0.0)
    o = jnp.dot(t, w2[...], preferred_element_type=jnp.float32) + b2[...]
    if relu_last:
        o = jnp.maximum(o, 0.0)
    out_ref[...] = o
    if with_gate:
        g = jnp.maximum(jnp.dot(o, wg1[...], preferred_element_type=jnp.float32)
                        + bg1[...], 0.0)
        gate_ref[...] = jnp.dot(g, wg2[...], preferred_element_type=jnp.float32) + bg2[...]
    else:
        gate_ref[...] = jnp.zeros_like(gate_ref)


def _mlp(h, parts, W1i, b1i, W2i, b2i, scale, Wg1, bg1, Wg2, bg2, relu_last,
         with_gate):
    full = lambda shape: pl.BlockSpec(shape, lambda i: (0,) * len(shape))
    out, gate = pl.pallas_call(
        functools.partial(_mlp_body, relu_last=relu_last, with_gate=with_gate),
        grid=(N // BN,),
        in_specs=[
            pl.BlockSpec((BN, D), lambda i: (i, 0)),
            pl.BlockSpec((1, BN, D), lambda i: (0, i, 0)),
            pl.BlockSpec((1, BN, D), lambda i: (1, i, 0)),
            full((D, D)), full((1, D)), full((D, D)), full((1, D)),
            full((1, 1)),
            full((D, D)), full((1, D)), full((D, 1)), full((1, 1)),
        ],
        out_specs=[pl.BlockSpec((BN, D), lambda i: (i, 0)),
                   pl.BlockSpec((BN, 1), lambda i: (i, 0))],
        out_shape=[jax.ShapeDtypeStruct((N, D), jnp.float32),
                   jax.ShapeDtypeStruct((N, 1), jnp.float32)],
    )(h, parts, parts, W1i, b1i.reshape(1, D), W2i, b2i.reshape(1, D),
      scale, Wg1, bg1.reshape(1, D), Wg2, bg2.reshape(1, 1))
    return out, gate


def _seg_max_body(gate_ref, batch_ref, out_ref):
    i = pl.program_id(0)
    mask = batch_ref[...] == lax.broadcasted_iota(jnp.int32, (BN, B), 1)
    vals = jnp.where(mask, gate_ref[...], -jnp.inf)
    cur = jnp.max(vals, axis=0, keepdims=True)

    @pl.when(i == 0)
    def _():
        out_ref[...] = cur

    @pl.when(i > 0)
    def _():
        out_ref[...] = jnp.maximum(out_ref[...], cur)


def _seg_max(gate, batch2d):
    return pl.pallas_call(
        _seg_max_body,
        grid=(N // BN,),
        in_specs=[pl.BlockSpec((BN, 1), lambda i: (i, 0)),
                  pl.BlockSpec((BN, 1), lambda i: (i, 0))],
        out_specs=pl.BlockSpec((1, B), lambda i: (0, 0)),
        out_shape=jax.ShapeDtypeStruct((1, B), jnp.float32),
    )(gate, batch2d)


def _seg_accum_body(gate_ref, batch_ref, gmax_ref, h_ref, num_ref, gsum_ref):
    i = pl.program_id(0)
    gm = gmax_ref[...]
    gm = jnp.where(jnp.isfinite(gm), gm, 0.0)
    mask = batch_ref[...] == lax.broadcasted_iota(jnp.int32, (BN, B), 1)
    gm_node = jnp.sum(jnp.where(mask, gm, 0.0), axis=1, keepdims=True)
    ge = jnp.exp(gate_ref[...] - gm_node)
    wmask = mask.astype(jnp.float32) * ge
    ones = jnp.ones((BN, 1), jnp.float32)
    gsum_cur = lax.dot_general(wmask, ones, (((0,), (0,)), ((), ())),
                               preferred_element_type=jnp.float32)
    num_cur = lax.dot_general(wmask, h_ref[...], (((0,), (0,)), ((), ())),
                              preferred_element_type=jnp.float32)

    @pl.when(i == 0)
    def _():
        num_ref[...] = num_cur
        gsum_ref[...] = gsum_cur

    @pl.when(i > 0)
    def _():
        num_ref[...] = num_ref[...] + num_cur
        gsum_ref[...] = gsum_ref[...] + gsum_cur


def _seg_accum(gate, batch2d, gmax, h):
    return pl.pallas_call(
        _seg_accum_body,
        grid=(N // BN,),
        in_specs=[pl.BlockSpec((BN, 1), lambda i: (i, 0)),
                  pl.BlockSpec((BN, 1), lambda i: (i, 0)),
                  pl.BlockSpec((1, B), lambda i: (0, 0)),
                  pl.BlockSpec((BN, D), lambda i: (i, 0))],
        out_specs=[pl.BlockSpec((B, D), lambda i: (0, 0)),
                   pl.BlockSpec((B, 1), lambda i: (0, 0))],
        out_shape=[jax.ShapeDtypeStruct((B, D), jnp.float32),
                   jax.ShapeDtypeStruct((B, 1), jnp.float32)],
    )(gate, batch2d, gmax, h)


def _head_body(num_ref, gsum_ref, il, ia, ib, iar, el, ea_, eb, ear,
               wp, wl, wa, wb, war, bl1_ref, wm, bm_ref, out_ref):
    pooled = num_ref[...] / (gsum_ref[...] + 1e-16)

    def onehot(idx_ref, k):
        return (idx_ref[...] == lax.broadcasted_iota(jnp.int32, (B, k), 1)
                ).astype(jnp.float32)

    dot = lambda a_, b_: jnp.dot(a_, b_, preferred_element_type=jnp.float32)
    z = dot(pooled, wp[...])
    z = z + dot(dot(onehot(il, 16), el[...]), wl[...])
    z = z + dot(dot(onehot(ia, 24), ea_[...]), wa[...])
    z = z + dot(dot(onehot(ib, 4), eb[...]), wb[...])
    z = z + dot(dot(onehot(iar, 16), ear[...]), war[...])
    z = jnp.maximum(z + bl1_ref[...], 0.0)
    out_ref[...] = dot(z, wm[...]) + bm_ref[...]


def _head(num, gsum, ligand_idx, additive_idx, base_idx, aryl_idx,
          emb_ligand, emb_additive, emb_base, emb_aryl, Wl1, bl1, Wm, bm):
    args = (num, gsum, ligand_idx.reshape(B, 1), additive_idx.reshape(B, 1),
            base_idx.reshape(B, 1), aryl_idx.reshape(B, 1),
            emb_ligand, emb_additive, emb_base, emb_aryl,
            Wl1[:D], Wl1[D:D + EMB], Wl1[D + EMB:D + 2 * EMB],
            Wl1[D + 2 * EMB:D + 3 * EMB], Wl1[D + 3 * EMB:],
            bl1.reshape(1, D), Wm, bm.reshape(1, 1))
    return pl.pallas_call(
        _head_body,
        out_shape=jax.ShapeDtypeStruct((B, 1), jnp.float32),
    )(*args)


def kernel(x, edge_index, batch, ligand_idx, additive_idx, base_idx, aryl_idx,
           edge_attr, W1, b1, W2, b2, We, be, eps, Wg1, bg1, Wg2, bg2,
           emb_ligand, emb_additive, emb_base, emb_aryl, Wl1, bl1, Wm, bm):
    src = edge_index[0].reshape(NW, NPHASE, 1, EPP)
    dst = edge_index[1].reshape(NW, NPHASE, CPP, C)
    zeros = jnp.zeros((N, D), jnp.float32)
    batch2d = batch.reshape(N, 1)
    scales = (1.0 + eps).reshape(3, 1, 1)

    sc_mp = _make_sc_message_pass()

    h = x
    gate = None
    for i in range(3):
        ea_i = _ea_one(edge_attr, We[i], be[i])
        parts = sc_mp(h, ea_i, src, dst, zeros)
        h, gate = _mlp(h, parts, W1[i], b1[i], W2[i], b2[i], scales[i],
                       Wg1, bg1, Wg2, bg2, relu_last=(i < 2),
                       with_gate=(i == 2))

    gmax = _seg_max(gate, batch2d)
    num, gsum = _seg_accum(gate, batch2d, gmax, h)
    return _head(num, gsum, ligand_idx, additive_idx, base_idx, aryl_idx,
                 emb_ligand, emb_additive, emb_base, emb_aryl, Wl1, bl1,
                 Wm, bm)
